# baseline (device time: 257703 ns/iter reference)
import jax
import jax.numpy as jnp
from jax import lax
from jax.experimental import pallas as pl
from jax.experimental.pallas import tpu as pltpu

N_SLICES = 8


def kernel(Q, K, V):
    b, kv, h, d = K.shape
    kv8 = kv // N_SLICES
    scale = d ** -0.5
    n_stage = 4

    def body(s_ref, q_ref, k_ref, v_ref, o_ref,
             o_scr, st_scr, o_rcv, st_rcv, send_sems, recv_sems):
        i = pl.program_id(0)
        my_x = lax.axis_index("x")
        my_y = lax.axis_index("y")
        my_z = lax.axis_index("z")
        partners = [
            (my_x, my_y, my_z ^ 1),
            (my_x, my_y, my_z ^ 2),
            (my_x, 1 - my_y, my_z),
            (1 - my_x, my_y, my_z),
        ]

        @pl.when(i == 0)
        def _():
            barrier = pltpu.get_barrier_semaphore()
            for p in partners:
                pl.semaphore_signal(
                    barrier, inc=1, device_id=p,
                    device_id_type=pl.DeviceIdType.MESH,
                )
            pl.semaphore_wait(barrier, n_stage)

        eyef = (
            lax.broadcasted_iota(jnp.int32, (h, h), 0)
            == lax.broadcasted_iota(jnp.int32, (h, h), 1)
        ).astype(jnp.float32)
        q = q_ref[0, 0].astype(jnp.bfloat16)
        k2d = k_ref[0].reshape(kv8 * h, d).astype(jnp.bfloat16)
        r = lax.dot_general(
            k2d, q, (((1,), (1,)), ((), ())),
            preferred_element_type=jnp.float32,
        )
        s3 = r.reshape(kv8, h, h) * eyef[None]
        s = jnp.sum(s3, axis=1) * scale
        m = jnp.max(s, axis=0, keepdims=True)
        p = jnp.exp(s - m)
        l = jnp.sum(p, axis=0, keepdims=True)
        p3 = (p[:, None, :] * eyef[None]).astype(jnp.bfloat16)
        p2d = p3.reshape(kv8 * h, h)
        v2d = v_ref[0].reshape(kv8 * h, d).astype(jnp.bfloat16)
        o = lax.dot_general(
            p2d, v2d, (((0,), (0,)), ((), ())),
            preferred_element_type=jnp.float32,
        )

        o_scr[pl.ds(i, 1)] = o[None]
        st_scr[pl.ds(i, 1), :] = m
        st_scr[pl.ds(b + i, 1), :] = l

        @pl.when(i == b - 1)
        def _():
            for st, partner in enumerate(partners):
                copy_o = pltpu.make_async_remote_copy(
                    src_ref=o_scr, dst_ref=o_rcv.at[st],
                    send_sem=send_sems.at[2 * st],
                    recv_sem=recv_sems.at[2 * st],
                    device_id=partner, device_id_type=pl.DeviceIdType.MESH,
                )
                copy_s = pltpu.make_async_remote_copy(
                    src_ref=st_scr, dst_ref=st_rcv.at[st],
                    send_sem=send_sems.at[2 * st + 1],
                    recv_sem=recv_sems.at[2 * st + 1],
                    device_id=partner, device_id_type=pl.DeviceIdType.MESH,
                )
                copy_o.start()
                copy_s.start()
                copy_o.wait()
                copy_s.wait()

                m_loc = st_scr[pl.ds(0, b), :]
                l_loc = st_scr[pl.ds(b, b), :]
                m_rem = st_rcv[st, pl.ds(0, b), :]
                l_rem = st_rcv[st, pl.ds(b, b), :]
                m_new = jnp.maximum(m_loc, m_rem)
                a_loc = jnp.exp(m_loc - m_new)
                a_rem = jnp.exp(m_rem - m_new)
                st_scr[pl.ds(0, b), :] = m_new
                st_scr[pl.ds(b, b), :] = l_loc * a_loc + l_rem * a_rem
                o_scr[...] = (
                    o_scr[...] * a_loc[:, :, None]
                    + o_rcv[st] * a_rem[:, :, None]
                )

            l_fin = st_scr[pl.ds(b, b), :]
            o_ref[...] = (o_scr[...] / l_fin[:, :, None])[:, None]

    grid_spec = pltpu.PrefetchScalarGridSpec(
        num_scalar_prefetch=1,
        grid=(b,),
        in_specs=[
            pl.BlockSpec((1, 1, h, d), lambda i, s: (i, 0, 0, 0)),
            pl.BlockSpec((1, kv8, h, d), lambda i, s: (i, s[0], 0, 0)),
            pl.BlockSpec((1, kv8, h, d), lambda i, s: (i, s[0], 0, 0)),
        ],
        out_specs=pl.BlockSpec((b, 1, h, d), lambda i, s: (0, 0, 0, 0)),
        scratch_shapes=[
            pltpu.VMEM((b, h, d), jnp.float32),
            pltpu.VMEM((2 * b, h), jnp.float32),
            pltpu.VMEM((n_stage, b, h, d), jnp.float32),
            pltpu.VMEM((n_stage, 2 * b, h), jnp.float32),
            pltpu.SemaphoreType.DMA((2 * n_stage,)),
            pltpu.SemaphoreType.DMA((2 * n_stage,)),
        ],
    )

    ridx = (lax.axis_index("y") * 4 + lax.axis_index("z")).astype(jnp.int32)
    return pl.pallas_call(
        body,
        grid_spec=grid_spec,
        out_shape=jax.ShapeDtypeStruct((b, 1, h, d), jnp.float32),
        compiler_params=pltpu.CompilerParams(
            collective_id=0,
            dimension_semantics=("arbitrary",),
            vmem_limit_bytes=64 * 1024 * 1024,
        ),
    )(jnp.reshape(ridx, (1,)), Q, K, V)


# device time: 38597 ns/iter; 6.6768x vs baseline; 6.6768x over previous
import jax
import jax.numpy as jnp
from jax import lax
from jax.experimental import pallas as pl
from jax.experimental.pallas import tpu as pltpu

N_SLICES = 8


def kernel(Q, K, V):
    b, kv, h, d = K.shape
    kv8 = kv // N_SLICES
    scale = d ** -0.5
    n_stage = 4

    def body(s_ref, q_ref, k_ref, v_ref, o_ref,
             o_scr, st_scr, o_rcv, st_rcv, send_sems, recv_sems):
        i = pl.program_id(0)
        my_x = lax.axis_index("x")
        my_y = lax.axis_index("y")
        my_z = lax.axis_index("z")
        partners = [
            (my_x, my_y, my_z ^ 1),
            (my_x, my_y, my_z ^ 2),
            (my_x, 1 - my_y, my_z),
            (1 - my_x, my_y, my_z),
        ]

        @pl.when(i == 0)
        def _():
            barrier = pltpu.get_barrier_semaphore()
            for p in partners:
                pl.semaphore_signal(
                    barrier, inc=1, device_id=p,
                    device_id_type=pl.DeviceIdType.MESH,
                )
            pl.semaphore_wait(barrier, n_stage)

        eyef = (
            lax.broadcasted_iota(jnp.int32, (h, h), 0)
            == lax.broadcasted_iota(jnp.int32, (h, h), 1)
        ).astype(jnp.float32)
        q = q_ref[0, 0]
        qbd = (q[:, None, :] * eyef[:, :, None]).reshape(h, h * d)
        k2 = k_ref[0].reshape(h * d, kv8).astype(jnp.bfloat16)
        s = lax.dot_general(
            qbd.astype(jnp.bfloat16), k2,
            (((1,), (0,)), ((), ())),
            preferred_element_type=jnp.float32,
        ) * scale
        m = jnp.max(s, axis=-1, keepdims=True)
        p = jnp.exp(s - m)
        l = jnp.sum(p, axis=-1, keepdims=True)
        v2 = v_ref[0].reshape(h * d, kv8).astype(jnp.bfloat16)
        r = lax.dot_general(
            p.astype(jnp.bfloat16), v2,
            (((1,), (1,)), ((), ())),
            preferred_element_type=jnp.float32,
        )
        o = jnp.sum(r.reshape(h, h, d) * eyef[:, :, None], axis=1)

        o_scr[pl.ds(i, 1)] = o[None]
        st_scr[pl.ds(i, 1), :] = m.reshape(1, h)
        st_scr[pl.ds(b + i, 1), :] = l.reshape(1, h)

        @pl.when(i == b - 1)
        def _():
            for st, partner in enumerate(partners):
                copy_o = pltpu.make_async_remote_copy(
                    src_ref=o_scr, dst_ref=o_rcv.at[st],
                    send_sem=send_sems.at[2 * st],
                    recv_sem=recv_sems.at[2 * st],
                    device_id=partner, device_id_type=pl.DeviceIdType.MESH,
                )
                copy_s = pltpu.make_async_remote_copy(
                    src_ref=st_scr, dst_ref=st_rcv.at[st],
                    send_sem=send_sems.at[2 * st + 1],
                    recv_sem=recv_sems.at[2 * st + 1],
                    device_id=partner, device_id_type=pl.DeviceIdType.MESH,
                )
                copy_o.start()
                copy_s.start()
                copy_o.wait()
                copy_s.wait()

                m_loc = st_scr[pl.ds(0, b), :]
                l_loc = st_scr[pl.ds(b, b), :]
                m_rem = st_rcv[st, pl.ds(0, b), :]
                l_rem = st_rcv[st, pl.ds(b, b), :]
                m_new = jnp.maximum(m_loc, m_rem)
                a_loc = jnp.exp(m_loc - m_new)
                a_rem = jnp.exp(m_rem - m_new)
                st_scr[pl.ds(0, b), :] = m_new
                st_scr[pl.ds(b, b), :] = l_loc * a_loc + l_rem * a_rem
                o_scr[...] = (
                    o_scr[...] * a_loc[:, :, None]
                    + o_rcv[st] * a_rem[:, :, None]
                )

            l_fin = st_scr[pl.ds(b, b), :]
            o_ref[...] = (o_scr[...] / l_fin[:, :, None])[:, None]

    grid_spec = pltpu.PrefetchScalarGridSpec(
        num_scalar_prefetch=1,
        grid=(b,),
        in_specs=[
            pl.BlockSpec((1, 1, h, d), lambda i, s: (i, 0, 0, 0)),
            pl.BlockSpec((1, h, d, kv8), lambda i, s: (i, 0, 0, s[0])),
            pl.BlockSpec((1, h, d, kv8), lambda i, s: (i, 0, 0, s[0])),
        ],
        out_specs=pl.BlockSpec((b, 1, h, d), lambda i, s: (0, 0, 0, 0)),
        scratch_shapes=[
            pltpu.VMEM((b, h, d), jnp.float32),
            pltpu.VMEM((2 * b, h), jnp.float32),
            pltpu.VMEM((n_stage, b, h, d), jnp.float32),
            pltpu.VMEM((n_stage, 2 * b, h), jnp.float32),
            pltpu.SemaphoreType.DMA((2 * n_stage,)),
            pltpu.SemaphoreType.DMA((2 * n_stage,)),
        ],
    )

    ridx = (lax.axis_index("y") * 4 + lax.axis_index("z")).astype(jnp.int32)
    return pl.pallas_call(
        body,
        grid_spec=grid_spec,
        out_shape=jax.ShapeDtypeStruct((b, 1, h, d), jnp.float32),
        compiler_params=pltpu.CompilerParams(
            collective_id=0,
            dimension_semantics=("arbitrary",),
            vmem_limit_bytes=64 * 1024 * 1024,
        ),
    )(
        jnp.reshape(ridx, (1,)),
        Q,
        K.transpose(0, 2, 3, 1),
        V.transpose(0, 2, 3, 1),
    )


# device time: 33054 ns/iter; 7.7964x vs baseline; 1.1677x over previous
import jax
import jax.numpy as jnp
from jax import lax
from jax.experimental import pallas as pl
from jax.experimental.pallas import tpu as pltpu

N_SLICES = 8
NB = 4


def kernel(Q, K, V):
    b, kv, h, d = K.shape
    kv8 = kv // N_SLICES
    scale = d ** -0.5
    n_stage = 4
    n_step = b // NB
    bh = NB * h

    def body(s_ref, q_ref, k_ref, v_ref, o_ref,
             acc, rcv, send_sems, recv_sems):
        i = pl.program_id(0)
        my_x = lax.axis_index("x")
        my_y = lax.axis_index("y")
        my_z = lax.axis_index("z")
        partners = [
            (my_x, my_y, my_z ^ 1),
            (my_x, my_y, my_z ^ 2),
            (my_x, 1 - my_y, my_z),
            (1 - my_x, my_y, my_z),
        ]

        @pl.when(i == 0)
        def _():
            barrier = pltpu.get_barrier_semaphore()
            for prt in partners:
                pl.semaphore_signal(
                    barrier, inc=1, device_id=prt,
                    device_id_type=pl.DeviceIdType.MESH,
                )
            pl.semaphore_wait(barrier, n_stage)

        eyef = (
            lax.broadcasted_iota(jnp.int32, (bh, bh), 0)
            == lax.broadcasted_iota(jnp.int32, (bh, bh), 1)
        ).astype(jnp.float32)
        q2 = q_ref[:, 0].reshape(bh, d)
        qbd = (q2[:, None, :] * eyef[:, :, None]).reshape(bh, bh * d)
        k2 = k_ref[...].reshape(bh * d, kv8).astype(jnp.bfloat16)
        s = lax.dot_general(
            qbd.astype(jnp.bfloat16), k2,
            (((1,), (0,)), ((), ())),
            preferred_element_type=jnp.float32,
        ) * scale
        m = jnp.max(s, axis=-1, keepdims=True)
        p = jnp.exp(s - m)
        l = jnp.sum(p, axis=-1, keepdims=True)
        v2 = v_ref[...].reshape(bh * d, kv8).astype(jnp.bfloat16)
        r = lax.dot_general(
            p.astype(jnp.bfloat16), v2,
            (((1,), (1,)), ((), ())),
            preferred_element_type=jnp.float32,
        )
        o = jnp.sum(r.reshape(bh, bh, d) * eyef[:, :, None], axis=1)

        acc[0, pl.ds(i * NB, NB)] = o.reshape(NB, h, d)
        acc[0, pl.ds(b, 1), pl.ds(i * NB, NB), pl.ds(0, h)] = (
            m.reshape(NB, h)[None]
        )
        acc[0, pl.ds(b, 1), pl.ds(i * NB, NB), pl.ds(h, h)] = (
            l.reshape(NB, h)[None]
        )

        @pl.when(i == n_step - 1)
        def _():
            rdmas = []
            for st, partner in enumerate(partners):
                cur, nxt = st % 2, (st + 1) % 2
                rdma = pltpu.make_async_remote_copy(
                    src_ref=acc.at[cur], dst_ref=rcv.at[st],
                    send_sem=send_sems.at[st], recv_sem=recv_sems.at[st],
                    device_id=partner, device_id_type=pl.DeviceIdType.MESH,
                )
                rdma.start()
                rdma.wait_recv()
                rdmas.append(rdma)
                if st >= 1:
                    rdmas[st - 1].wait_send()

                slab_l = acc[cur, b]
                slab_r = rcv[st, b]
                m_loc, l_loc = slab_l[:, :h], slab_l[:, h:2 * h]
                m_rem, l_rem = slab_r[:, :h], slab_r[:, h:2 * h]
                m_new = jnp.maximum(m_loc, m_rem)
                a_loc = jnp.exp(m_loc - m_new)
                a_rem = jnp.exp(m_rem - m_new)
                l_new = l_loc * a_loc + l_rem * a_rem
                acc[nxt, pl.ds(0, b)] = (
                    acc[cur, pl.ds(0, b)] * a_loc[:, :, None]
                    + rcv[st, pl.ds(0, b)] * a_rem[:, :, None]
                )
                acc[nxt, pl.ds(b, 1)] = jnp.concatenate(
                    [m_new, l_new, jnp.zeros((b, d - 2 * h), jnp.float32)],
                    axis=1,
                )[None]

            rdmas[n_stage - 1].wait_send()
            fin = n_stage % 2
            l_fin = acc[fin, b][:, h:2 * h]
            o_ref[...] = (acc[fin, pl.ds(0, b)] / l_fin[:, :, None])[:, None]

    grid_spec = pltpu.PrefetchScalarGridSpec(
        num_scalar_prefetch=1,
        grid=(n_step,),
        in_specs=[
            pl.BlockSpec((NB, 1, h, d), lambda i, s: (i, 0, 0, 0)),
            pl.BlockSpec((NB, h, d, kv8), lambda i, s: (i, 0, 0, s[0])),
            pl.BlockSpec((NB, h, d, kv8), lambda i, s: (i, 0, 0, s[0])),
        ],
        out_specs=pl.BlockSpec((b, 1, h, d), lambda i, s: (0, 0, 0, 0)),
        scratch_shapes=[
            pltpu.VMEM((2, b + 1, h, d), jnp.float32),
            pltpu.VMEM((n_stage, b + 1, h, d), jnp.float32),
            pltpu.SemaphoreType.DMA((n_stage,)),
            pltpu.SemaphoreType.DMA((n_stage,)),
        ],
    )

    ridx = (lax.axis_index("y") * 4 + lax.axis_index("z")).astype(jnp.int32)
    return pl.pallas_call(
        body,
        grid_spec=grid_spec,
        out_shape=jax.ShapeDtypeStruct((b, 1, h, d), jnp.float32),
        compiler_params=pltpu.CompilerParams(
            collective_id=0,
            dimension_semantics=("arbitrary",),
            vmem_limit_bytes=64 * 1024 * 1024,
        ),
    )(
        jnp.reshape(ridx, (1,)),
        Q,
        K.transpose(0, 2, 3, 1),
        V.transpose(0, 2, 3, 1),
    )


# device time: 32023 ns/iter; 8.0474x vs baseline; 1.0322x over previous
import jax
import jax.numpy as jnp
from jax import lax
from jax.experimental import pallas as pl
from jax.experimental.pallas import tpu as pltpu

N_SLICES = 8
NB = 4
NW = 8


def kernel(Q, K, V):
    b, kv, h, d = K.shape
    kv8 = kv // N_SLICES
    scale = d ** -0.5
    n_stage = 4
    n_step = b // NB
    bh = NB * h

    def body(s_ref, q_ref, k_ref, v_ref, o_ref,
             acc0, acc1, rcv0, rcv1, ss0, rs0, ss1, rs1):
        i = pl.program_id(0)
        my_x = lax.axis_index("x")
        my_y = lax.axis_index("y")
        my_z = lax.axis_index("z")
        partners = [
            (my_x, my_y, my_z ^ 1),
            (my_x, my_y, my_z ^ 2),
            (my_x, 1 - my_y, my_z),
            (1 - my_x, my_y, my_z),
        ]
        waves = [(acc0, rcv0, ss0, rs0), (acc1, rcv1, ss1, rs1)]

        def rdma_for(w, st):
            acc, rcv, ssem, rsem = waves[w]
            return pltpu.make_async_remote_copy(
                src_ref=acc.at[st % 2], dst_ref=rcv.at[st],
                send_sem=ssem.at[st], recv_sem=rsem.at[st],
                device_id=partners[st], device_id_type=pl.DeviceIdType.MESH,
            )

        def start_stage(w, st):
            rdma_for(w, st).start()

        def finish_stage(w, st):
            acc, rcv, _, _ = waves[w]
            rdma_for(w, st).wait_recv()
            if st >= 1:
                rdma_for(w, st - 1).wait_send()
            cur, nxt = st % 2, (st + 1) % 2
            slab_l = acc[cur, NW]
            slab_r = rcv[st, NW]
            m_loc, l_loc = slab_l[:, :h], slab_l[:, h:2 * h]
            m_rem, l_rem = slab_r[:, :h], slab_r[:, h:2 * h]
            m_new = jnp.maximum(m_loc, m_rem)
            a_loc = jnp.exp(m_loc - m_new)
            a_rem = jnp.exp(m_rem - m_new)
            l_new = l_loc * a_loc + l_rem * a_rem
            acc[nxt, pl.ds(0, NW)] = (
                acc[cur, pl.ds(0, NW)] * a_loc[:NW, :, None]
                + rcv[st, pl.ds(0, NW)] * a_rem[:NW, :, None]
            )
            acc[nxt, pl.ds(NW, 1)] = jnp.concatenate(
                [m_new, l_new, jnp.zeros((h, d - 2 * h), jnp.float32)],
                axis=1,
            )[None]

        @pl.when(i == 0)
        def _():
            barrier = pltpu.get_barrier_semaphore()
            for prt in partners:
                pl.semaphore_signal(
                    barrier, inc=1, device_id=prt,
                    device_id_type=pl.DeviceIdType.MESH,
                )
            pl.semaphore_wait(barrier, n_stage)

        eyef = (
            lax.broadcasted_iota(jnp.int32, (bh, bh), 0)
            == lax.broadcasted_iota(jnp.int32, (bh, bh), 1)
        ).astype(jnp.float32)
        q2 = q_ref[:, 0].reshape(bh, d)
        qbd = (q2[:, None, :] * eyef[:, :, None]).reshape(bh, bh * d)
        k2 = k_ref[...].reshape(bh * d, kv8).astype(jnp.bfloat16)
        s = lax.dot_general(
            qbd.astype(jnp.bfloat16), k2,
            (((1,), (0,)), ((), ())),
            preferred_element_type=jnp.float32,
        ) * scale
        m = jnp.max(s, axis=-1, keepdims=True)
        p = jnp.exp(s - m)
        l = jnp.sum(p, axis=-1, keepdims=True)
        v2 = v_ref[...].reshape(bh * d, kv8).astype(jnp.bfloat16)
        r = lax.dot_general(
            p.astype(jnp.bfloat16), v2,
            (((1,), (1,)), ((), ())),
            preferred_element_type=jnp.float32,
        )
        o = jnp.sum(r.reshape(bh, bh, d) * eyef[:, :, None], axis=1)

        for w in range(2):
            @pl.when(i // 2 == w)
            def _(w=w):
                acc, _, _, _ = waves[w]
                off = (i % 2) * NB
                acc[0, pl.ds(off, NB)] = o.reshape(NB, h, d)
                acc[0, pl.ds(NW, 1), pl.ds(off, NB), pl.ds(0, h)] = (
                    m.reshape(NB, h)[None]
                )
                acc[0, pl.ds(NW, 1), pl.ds(off, NB), pl.ds(h, h)] = (
                    l.reshape(NB, h)[None]
                )

        @pl.when(i == 1)
        def _():
            start_stage(0, 0)

        @pl.when(i == 2)
        def _():
            finish_stage(0, 0)
            start_stage(0, 1)

        @pl.when(i == n_step - 1)
        def _():
            finish_stage(0, 1)
            start_stage(0, 2)
            start_stage(1, 0)
            finish_stage(0, 2)
            start_stage(0, 3)
            finish_stage(1, 0)
            start_stage(1, 1)
            finish_stage(0, 3)
            finish_stage(1, 1)
            start_stage(1, 2)
            finish_stage(1, 2)
            start_stage(1, 3)
            finish_stage(1, 3)
            rdma_for(0, 3).wait_send()
            rdma_for(1, 3).wait_send()

            l0 = acc0[0, NW][:, h:2 * h]
            l1 = acc1[0, NW][:, h:2 * h]
            o_ref[pl.ds(0, NW)] = (
                acc0[0, pl.ds(0, NW)] / l0[:NW, :, None]
            )[:, None]
            o_ref[pl.ds(NW, NW)] = (
                acc1[0, pl.ds(0, NW)] / l1[:NW, :, None]
            )[:, None]

    grid_spec = pltpu.PrefetchScalarGridSpec(
        num_scalar_prefetch=1,
        grid=(n_step,),
        in_specs=[
            pl.BlockSpec((NB, 1, h, d), lambda i, s: (i, 0, 0, 0)),
            pl.BlockSpec((NB, h, d, kv8), lambda i, s: (i, 0, 0, s[0])),
            pl.BlockSpec((NB, h, d, kv8), lambda i, s: (i, 0, 0, s[0])),
        ],
        out_specs=pl.BlockSpec((b, 1, h, d), lambda i, s: (0, 0, 0, 0)),
        scratch_shapes=[
            pltpu.VMEM((2, NW + 1, h, d), jnp.float32),
            pltpu.VMEM((2, NW + 1, h, d), jnp.float32),
            pltpu.VMEM((n_stage, NW + 1, h, d), jnp.float32),
            pltpu.VMEM((n_stage, NW + 1, h, d), jnp.float32),
            pltpu.SemaphoreType.DMA((n_stage,)),
            pltpu.SemaphoreType.DMA((n_stage,)),
            pltpu.SemaphoreType.DMA((n_stage,)),
            pltpu.SemaphoreType.DMA((n_stage,)),
        ],
    )

    ridx = (lax.axis_index("y") * 4 + lax.axis_index("z")).astype(jnp.int32)
    return pl.pallas_call(
        body,
        grid_spec=grid_spec,
        out_shape=jax.ShapeDtypeStruct((b, 1, h, d), jnp.float32),
        compiler_params=pltpu.CompilerParams(
            collective_id=0,
            dimension_semantics=("arbitrary",),
            vmem_limit_bytes=64 * 1024 * 1024,
        ),
    )(
        jnp.reshape(ridx, (1,)),
        Q,
        K.transpose(0, 2, 3, 1),
        V.transpose(0, 2, 3, 1),
    )


# device time: 21174 ns/iter; 12.1707x vs baseline; 1.5124x over previous
import jax
import jax.numpy as jnp
from jax import lax
from jax.experimental import pallas as pl
from jax.experimental.pallas import tpu as pltpu

N_SLICES = 8
NB = 4


def kernel(Q, K, V):
    b, kv, h, d = K.shape
    hh = h // N_SLICES
    scale = d ** -0.5
    n_step = b // NB
    rows = NB * hh

    deltas = [
        (dy, dz) for dy in (0, 1) for dz in (0, 1, 2, 3) if (dy, dz) != (0, 0)
    ]

    def body(s_ref, q_ref, k_ref, v_ref, o_ref,
             axc, g, xs_sem, xr_sem, gs_sem, gr_sem):
        i = pl.program_id(0)
        my_x = lax.axis_index("x")
        my_y = lax.axis_index("y")
        my_z = lax.axis_index("z")
        x_peer = (1 - my_x, my_y, my_z)
        g_peers = [(my_x, my_y ^ dy, (my_z + dz) % 4) for dy, dz in deltas]
        off = s_ref[0] * hh

        @pl.when(i == 0)
        def _():
            barrier = pltpu.get_barrier_semaphore()
            for prt in [x_peer] + g_peers:
                pl.semaphore_signal(
                    barrier, inc=1, device_id=prt,
                    device_id_type=pl.DeviceIdType.MESH,
                )
            pl.semaphore_wait(barrier, 1 + len(g_peers))

        eyef = (
            lax.broadcasted_iota(jnp.int32, (rows, rows), 0)
            == lax.broadcasted_iota(jnp.int32, (rows, rows), 1)
        ).astype(jnp.float32)
        q2 = q_ref[:, 0, pl.ds(off, hh), :].reshape(rows, d)
        qbd = (q2[:, None, :] * eyef[:, :, None]).reshape(rows, rows * d)
        k2 = k_ref[...].reshape(rows * d, kv).astype(jnp.bfloat16)
        s = lax.dot_general(
            qbd.astype(jnp.bfloat16), k2,
            (((1,), (0,)), ((), ())),
            preferred_element_type=jnp.float32,
        ) * scale
        m = jnp.max(s, axis=-1, keepdims=True)
        p = jnp.exp(s - m)
        l = jnp.sum(p, axis=-1, keepdims=True)
        v2 = v_ref[...].reshape(rows * d, kv).astype(jnp.bfloat16)
        r = lax.dot_general(
            p.astype(jnp.bfloat16), v2,
            (((1,), (1,)), ((), ())),
            preferred_element_type=jnp.float32,
        )
        o = jnp.sum(r.reshape(rows, rows, d) * eyef[:, :, None], axis=1)

        axc[0, pl.ds(i * NB, NB)] = o.reshape(NB, hh, d)
        for w in range(n_step):
            @pl.when(i == w)
            def _(w=w):
                axc[0, pl.ds(b, 1), :, pl.ds(w * NB, NB)] = (
                    m.reshape(NB, hh).T[None]
                )
                axc[0, pl.ds(b, 1), :, pl.ds(b + w * NB, NB)] = (
                    l.reshape(NB, hh).T[None]
                )

        @pl.when(i == n_step - 1)
        def _():
            x_rdma = pltpu.make_async_remote_copy(
                src_ref=axc.at[0], dst_ref=axc.at[1],
                send_sem=xs_sem.at[0], recv_sem=xr_sem.at[0],
                device_id=x_peer, device_id_type=pl.DeviceIdType.MESH,
            )
            x_rdma.start()
            x_rdma.wait_recv()

            slab_l = axc[0, b]
            slab_r = axc[1, b]
            m_loc, l_loc = slab_l[:, :b], slab_l[:, b:2 * b]
            m_rem, l_rem = slab_r[:, :b], slab_r[:, b:2 * b]
            m_new = jnp.maximum(m_loc, m_rem)
            a_loc = jnp.exp(m_loc - m_new).T[:, :, None]
            a_rem = jnp.exp(m_rem - m_new).T[:, :, None]
            l_new = (l_loc * jnp.exp(m_loc - m_new)
                     + l_rem * jnp.exp(m_rem - m_new)).T[:, :, None]
            o_fin = (
                axc[0, pl.ds(0, b)] * a_loc + axc[1, pl.ds(0, b)] * a_rem
            ) / l_new
            g[pl.ds(off, hh)] = o_fin.transpose(1, 0, 2)

            for j, prt in enumerate(g_peers):
                pltpu.make_async_remote_copy(
                    src_ref=g.at[pl.ds(off, hh)],
                    dst_ref=g.at[pl.ds(off, hh)],
                    send_sem=gs_sem.at[j], recv_sem=gr_sem.at[j],
                    device_id=prt, device_id_type=pl.DeviceIdType.MESH,
                ).start()
            for j, prt in enumerate(g_peers):
                pltpu.make_async_remote_copy(
                    src_ref=g.at[pl.ds(0, hh)],
                    dst_ref=g.at[pl.ds(0, hh)],
                    send_sem=gs_sem.at[j], recv_sem=gr_sem.at[j],
                    device_id=prt, device_id_type=pl.DeviceIdType.MESH,
                ).wait_recv()
            for j, prt in enumerate(g_peers):
                pltpu.make_async_remote_copy(
                    src_ref=g.at[pl.ds(off, hh)],
                    dst_ref=g.at[pl.ds(off, hh)],
                    send_sem=gs_sem.at[j], recv_sem=gr_sem.at[j],
                    device_id=prt, device_id_type=pl.DeviceIdType.MESH,
                ).wait_send()
            x_rdma.wait_send()

            o_ref[...] = g[...].transpose(1, 0, 2)[:, None]

    grid_spec = pltpu.PrefetchScalarGridSpec(
        num_scalar_prefetch=1,
        grid=(n_step,),
        in_specs=[
            pl.BlockSpec((NB, 1, h, d), lambda i, s: (i, 0, 0, 0)),
            pl.BlockSpec((NB, hh, d, kv), lambda i, s: (i, s[0], 0, 0)),
            pl.BlockSpec((NB, hh, d, kv), lambda i, s: (i, s[0], 0, 0)),
        ],
        out_specs=pl.BlockSpec((b, 1, h, d), lambda i, s: (0, 0, 0, 0)),
        scratch_shapes=[
            pltpu.VMEM((2, b + 1, hh, d), jnp.float32),
            pltpu.VMEM((h, b, d), jnp.float32),
            pltpu.SemaphoreType.DMA((1,)),
            pltpu.SemaphoreType.DMA((1,)),
            pltpu.SemaphoreType.DMA((len(deltas),)),
            pltpu.SemaphoreType.DMA((len(deltas),)),
        ],
    )

    ridx = (lax.axis_index("y") * 4 + lax.axis_index("z")).astype(jnp.int32)
    return pl.pallas_call(
        body,
        grid_spec=grid_spec,
        out_shape=jax.ShapeDtypeStruct((b, 1, h, d), jnp.float32),
        compiler_params=pltpu.CompilerParams(
            collective_id=0,
            dimension_semantics=("arbitrary",),
            vmem_limit_bytes=64 * 1024 * 1024,
        ),
    )(
        jnp.reshape(ridx, (1,)),
        Q,
        K.transpose(0, 2, 3, 1),
        V.transpose(0, 2, 3, 1),
    )


# device time: 21125 ns/iter; 12.1990x vs baseline; 1.0023x over previous
import jax
import jax.numpy as jnp
from jax import lax
from jax.experimental import pallas as pl
from jax.experimental.pallas import tpu as pltpu

N_SLICES = 8
NB = 4


def kernel(Q, K, V):
    b, kv, h, d = K.shape
    hh = h // N_SLICES
    scale = d ** -0.5
    n_step = b // NB
    rows = NB * hh

    deltas = [
        (dy, dz) for dy in (0, 1) for dz in (0, 1, 2, 3) if (dy, dz) != (0, 0)
    ]

    def body(s_ref, q_ref, k_ref, v_ref, o_ref,
             axc, g, xs_sem, xr_sem, gs_sem, gr_sem):
        i = pl.program_id(0)
        my_x = lax.axis_index("x")
        my_y = lax.axis_index("y")
        my_z = lax.axis_index("z")
        x_peer = (1 - my_x, my_y, my_z)
        g_peers = [(my_x, my_y ^ dy, (my_z + dz) % 4) for dy, dz in deltas]
        off = s_ref[0] * hh

        @pl.when(i == 0)
        def _():
            barrier = pltpu.get_barrier_semaphore()
            for prt in [x_peer] + g_peers:
                pl.semaphore_signal(
                    barrier, inc=1, device_id=prt,
                    device_id_type=pl.DeviceIdType.MESH,
                )
            pl.semaphore_wait(barrier, 1 + len(g_peers))

        eyef = (
            lax.broadcasted_iota(jnp.int32, (rows, rows), 0)
            == lax.broadcasted_iota(jnp.int32, (rows, rows), 1)
        ).astype(jnp.float32)
        q2 = q_ref[:, 0, pl.ds(off, hh), :].reshape(rows, d)
        qbd = (q2[:, None, :] * eyef[:, :, None]).reshape(rows, rows * d)
        k2 = k_ref[...].reshape(rows * d, kv).astype(jnp.bfloat16)
        s = lax.dot_general(
            qbd.astype(jnp.bfloat16), k2,
            (((1,), (0,)), ((), ())),
            preferred_element_type=jnp.float32,
        ) * scale
        m = jnp.max(s, axis=-1, keepdims=True)
        p = jnp.exp(s - m)
        l = jnp.sum(p, axis=-1, keepdims=True)
        v2 = v_ref[...].reshape(rows * d, kv).astype(jnp.bfloat16)
        r = lax.dot_general(
            p.astype(jnp.bfloat16), v2,
            (((1,), (1,)), ((), ())),
            preferred_element_type=jnp.float32,
        )
        o = jnp.sum(r.reshape(rows, rows, d) * eyef[:, :, None], axis=1)

        axc[0, pl.ds(i * NB, NB)] = o.reshape(NB, hh, d)
        for w in range(n_step):
            @pl.when(i == w)
            def _(w=w):
                axc[0, pl.ds(b, 1), :, pl.ds(2 * NB * w, NB)] = (
                    m.reshape(NB, hh).T[None]
                )
                axc[0, pl.ds(b, 1), :, pl.ds(2 * NB * w + NB, NB)] = (
                    l.reshape(NB, hh).T[None]
                )
                pltpu.make_async_remote_copy(
                    src_ref=axc.at[0, pl.ds(w * NB, NB)],
                    dst_ref=axc.at[1, pl.ds(w * NB, NB)],
                    send_sem=xs_sem.at[w], recv_sem=xr_sem.at[w],
                    device_id=x_peer, device_id_type=pl.DeviceIdType.MESH,
                ).start()
                if w == n_step - 1:
                    pltpu.make_async_remote_copy(
                        src_ref=axc.at[0, pl.ds(b, 1)],
                        dst_ref=axc.at[1, pl.ds(b, 1)],
                        send_sem=xs_sem.at[n_step], recv_sem=xr_sem.at[n_step],
                        device_id=x_peer, device_id_type=pl.DeviceIdType.MESH,
                    ).start()

        @pl.when(i == n_step - 1)
        def _():
            for w in range(n_step):
                pltpu.make_async_remote_copy(
                    src_ref=axc.at[0, pl.ds(w * NB, NB)],
                    dst_ref=axc.at[1, pl.ds(w * NB, NB)],
                    send_sem=xs_sem.at[w], recv_sem=xr_sem.at[w],
                    device_id=x_peer, device_id_type=pl.DeviceIdType.MESH,
                ).wait_recv()
            pltpu.make_async_remote_copy(
                src_ref=axc.at[0, pl.ds(b, 1)],
                dst_ref=axc.at[1, pl.ds(b, 1)],
                send_sem=xs_sem.at[n_step], recv_sem=xr_sem.at[n_step],
                device_id=x_peer, device_id_type=pl.DeviceIdType.MESH,
            ).wait_recv()

            def unpack(slab):
                ms = jnp.concatenate(
                    [slab[:, 2 * NB * w: 2 * NB * w + NB] for w in range(n_step)],
                    axis=1,
                )
                ls = jnp.concatenate(
                    [slab[:, 2 * NB * w + NB: 2 * NB * (w + 1)] for w in range(n_step)],
                    axis=1,
                )
                return ms, ls

            m_loc, l_loc = unpack(axc[0, b])
            m_rem, l_rem = unpack(axc[1, b])
            m_new = jnp.maximum(m_loc, m_rem)
            a_loc = jnp.exp(m_loc - m_new).T[:, :, None]
            a_rem = jnp.exp(m_rem - m_new).T[:, :, None]
            l_new = (l_loc * jnp.exp(m_loc - m_new)
                     + l_rem * jnp.exp(m_rem - m_new)).T[:, :, None]
            o_fin = (
                axc[0, pl.ds(0, b)] * a_loc + axc[1, pl.ds(0, b)] * a_rem
            ) / l_new
            g[pl.ds(off, hh)] = o_fin.transpose(1, 0, 2)

            for j, prt in enumerate(g_peers):
                pltpu.make_async_remote_copy(
                    src_ref=g.at[pl.ds(off, hh)],
                    dst_ref=g.at[pl.ds(off, hh)],
                    send_sem=gs_sem.at[j], recv_sem=gr_sem.at[j],
                    device_id=prt, device_id_type=pl.DeviceIdType.MESH,
                ).start()
            for j, prt in enumerate(g_peers):
                pltpu.make_async_remote_copy(
                    src_ref=g.at[pl.ds(0, hh)],
                    dst_ref=g.at[pl.ds(0, hh)],
                    send_sem=gs_sem.at[j], recv_sem=gr_sem.at[j],
                    device_id=prt, device_id_type=pl.DeviceIdType.MESH,
                ).wait_recv()
            for j, prt in enumerate(g_peers):
                pltpu.make_async_remote_copy(
                    src_ref=g.at[pl.ds(off, hh)],
                    dst_ref=g.at[pl.ds(off, hh)],
                    send_sem=gs_sem.at[j], recv_sem=gr_sem.at[j],
                    device_id=prt, device_id_type=pl.DeviceIdType.MESH,
                ).wait_send()
            for w in range(n_step):
                pltpu.make_async_remote_copy(
                    src_ref=axc.at[0, pl.ds(w * NB, NB)],
                    dst_ref=axc.at[1, pl.ds(w * NB, NB)],
                    send_sem=xs_sem.at[w], recv_sem=xr_sem.at[w],
                    device_id=x_peer, device_id_type=pl.DeviceIdType.MESH,
                ).wait_send()
            pltpu.make_async_remote_copy(
                src_ref=axc.at[0, pl.ds(b, 1)],
                dst_ref=axc.at[1, pl.ds(b, 1)],
                send_sem=xs_sem.at[n_step], recv_sem=xr_sem.at[n_step],
                device_id=x_peer, device_id_type=pl.DeviceIdType.MESH,
            ).wait_send()

            o_ref[...] = g[...].transpose(1, 0, 2)[:, None]

    grid_spec = pltpu.PrefetchScalarGridSpec(
        num_scalar_prefetch=1,
        grid=(n_step,),
        in_specs=[
            pl.BlockSpec((NB, 1, h, d), lambda i, s: (i, 0, 0, 0)),
            pl.BlockSpec((NB, hh, d, kv), lambda i, s: (i, s[0], 0, 0)),
            pl.BlockSpec((NB, hh, d, kv), lambda i, s: (i, s[0], 0, 0)),
        ],
        out_specs=pl.BlockSpec((b, 1, h, d), lambda i, s: (0, 0, 0, 0)),
        scratch_shapes=[
            pltpu.VMEM((2, b + 1, hh, d), jnp.float32),
            pltpu.VMEM((h, b, d), jnp.float32),
            pltpu.SemaphoreType.DMA((n_step + 1,)),
            pltpu.SemaphoreType.DMA((n_step + 1,)),
            pltpu.SemaphoreType.DMA((len(deltas),)),
            pltpu.SemaphoreType.DMA((len(deltas),)),
        ],
    )

    ridx = (lax.axis_index("y") * 4 + lax.axis_index("z")).astype(jnp.int32)
    return pl.pallas_call(
        body,
        grid_spec=grid_spec,
        out_shape=jax.ShapeDtypeStruct((b, 1, h, d), jnp.float32),
        compiler_params=pltpu.CompilerParams(
            collective_id=0,
            dimension_semantics=("arbitrary",),
            vmem_limit_bytes=64 * 1024 * 1024,
        ),
    )(
        jnp.reshape(ridx, (1,)),
        Q,
        K.transpose(0, 2, 3, 1),
        V.transpose(0, 2, 3, 1),
    )
